# baseline (speedup 1.0000x reference)
import functools

import jax
import jax.numpy as jnp
import numpy as np
from jax import lax
from jax.experimental import pallas as pl
from jax.experimental.pallas import tpu as pltpu
from jax.experimental.pallas import tpu_sc as plsc

NC, NS, L = 2, 16, 16
NW = NC * NS
B, SEQ, D = 1024, 200, 128
V = 50176
BPW = B // NW
CHA, CHB = 104, 96
DW = D // 4
ND = DW // L

_mesh = plsc.VectorSubcoreMesh(
    core_axis_name="c", subcore_axis_name="s", num_cores=NC, num_subcores=NS
)


@functools.partial(
    pl.kernel,
    out_type=jax.ShapeDtypeStruct((B, D), jnp.float32),
    mesh=_mesh,
    compiler_params=pltpu.CompilerParams(use_tc_tiling_on_sc=False),
    scratch_types=[
        pltpu.VMEM((BPW, SEQ), jnp.int32),
        pltpu.VMEM((CHA, DW), jnp.int32),
        pltpu.VMEM((CHB, DW), jnp.int32),
        pltpu.VMEM((BPW, D), jnp.float32),
        pltpu.SemaphoreType.DMA,
        pltpu.SemaphoreType.DMA,
    ],
)
def _encode(x_hbm, table_hbm, out_hbm, idx_v, rowsa, rowsb, out_v, sema, semb):
    wid = lax.axis_index("s") * NC + lax.axis_index("c")

    pltpu.sync_copy(x_hbm.at[pl.ds(wid * BPW, BPW)], idx_v)

    zero8 = tuple(jnp.zeros((L,), jnp.int32) for _ in range(ND))

    def fire_a(s):
        pltpu.async_copy(table_hbm.at[idx_v.at[s, pl.ds(0, CHA)]], rowsa, sema)

    def fire_b(s):
        pltpu.async_copy(table_hbm.at[idx_v.at[s, pl.ds(CHA, CHB)]], rowsb, semb)

    def wait_a(s):
        pltpu.make_async_copy(
            table_hbm.at[idx_v.at[s, pl.ds(0, CHA)]], rowsa, sema).wait()

    def wait_b(s):
        pltpu.make_async_copy(
            table_hbm.at[idx_v.at[s, pl.ds(CHA, CHB)]], rowsb, semb).wait()

    def reduce_chunk(buf, n, acc8):
        def row_body(r, a):
            return tuple(a[h] + buf[r, pl.ds(h * L, L)] for h in range(ND))

        return lax.fori_loop(0, n, row_body, acc8)

    fire_a(0)
    fire_b(0)

    def sample_body(s, carry):
        wait_a(s)
        acc8 = reduce_chunk(rowsa, CHA, zero8)

        @pl.when(s + 1 < BPW)
        def _():
            fire_a(s + 1)

        wait_b(s)
        acc8 = reduce_chunk(rowsb, CHB, acc8)

        @pl.when(s + 1 < BPW)
        def _():
            fire_b(s + 1)

        for h in range(ND):
            for k in range(4):
                neg = (acc8[h] >> (8 * k)) & 0xFF
                sv = SEQ - 2 * neg
                out_v[s, pl.ds(32 * k + 16 * h, L)] = sv.astype(jnp.float32)
        return carry

    lax.fori_loop(0, BPW, sample_body, 0)
    pltpu.sync_copy(out_v, out_hbm.at[pl.ds(wid * BPW, BPW)])


def kernel(x, table):
    x2 = x.astype(jnp.int32)
    sgn = jax.lax.shift_right_logical(
        jax.lax.bitcast_convert_type(table, jnp.int32), 31
    )
    q = [sgn[:, 32 * k : 32 * k + 32] for k in range(4)]
    tw = q[0] | (q[1] << 8) | (q[2] << 16) | (q[3] << 24)
    return _encode(x2, tw)
